# Initial kernel scaffold; baseline (speedup 1.0000x reference)
import jax
import jax.numpy as jnp
from jax.experimental import pallas as pl


def kernel(features, logits, W_fc, b_fc):
    raise NotImplementedError("write your pallas kernel here")


# speedup vs baseline: 26.3694x; 26.3694x over previous
import functools

import jax
import jax.numpy as jnp
import numpy as np
from jax.experimental import pallas as pl
from jax.experimental.pallas import tpu as pltpu


def _ash_block_kernel(x_ref, wt_ref, b_ref, o_ref, *, k: int):
    x = x_ref[...]
    R, D = x.shape

    fill = jnp.sum(x, axis=1, keepdims=True) * (1.0 / k)

    s = jax.lax.bitcast_convert_type(x, jnp.int32)
    key = jnp.where(s < 0, s ^ jnp.int32(0x7FFFFFFF), s)

    K = jnp.full((R, 1), np.int32(-2147483648))
    for bit in range(31, -1, -1):
        t = K + np.int32(np.uint32(1 << bit))
        cnt = jnp.sum((key >= t).astype(jnp.int32), axis=1, keepdims=True)
        K = jnp.where(cnt >= k, t, K)

    gt = key > K
    eq = key == K
    cnt_gt = jnp.sum(gt.astype(jnp.int32), axis=1, keepdims=True)
    cnt_eq = jnp.sum(eq.astype(jnp.int32), axis=1, keepdims=True)
    need = k - cnt_gt

    idx = jax.lax.broadcasted_iota(jnp.int32, (R, D), 1)
    m = cnt_eq - need + 1
    J = jnp.zeros((R, 1), jnp.int32)
    for bit in range(10, -1, -1):
        t = J | np.int32(1 << bit)
        cnt = jnp.sum((eq & (idx >= t)).astype(jnp.int32), axis=1,
                      keepdims=True)
        J = jnp.where(cnt >= m, t, J)

    mask = gt | (eq & (idx <= J))
    maskf = mask.astype(jnp.float32)

    colsum = jnp.dot(maskf, wt_ref[...],
                     preferred_element_type=jnp.float32)
    logits = fill * colsum + b_ref[...]

    mx = jnp.max(logits, axis=1, keepdims=True)
    lse = mx + jnp.log(jnp.sum(jnp.exp(logits - mx), axis=1, keepdims=True))
    o_ref[...] = -lse


@jax.jit
def kernel(features, logits, W_fc, b_fc):
    del logits
    B, D = features.shape
    C = W_fc.shape[0]
    k = D - int(round(D * 90 / 100.0))

    R = min(256, B)
    wt = W_fc.T
    b2 = b_fc.reshape(1, C)

    out = pl.pallas_call(
        functools.partial(_ash_block_kernel, k=k),
        grid=(B // R,),
        in_specs=[
            pl.BlockSpec((R, D), lambda i: (i, 0)),
            pl.BlockSpec((D, C), lambda i: (0, 0)),
            pl.BlockSpec((1, C), lambda i: (0, 0)),
        ],
        out_specs=pl.BlockSpec((R, 1), lambda i: (i, 0)),
        out_shape=jax.ShapeDtypeStruct((B, 1), jnp.float32),
    )(features, wt, b2)
    return out


# speedup vs baseline: 37.3452x; 1.4162x over previous
import functools

import jax
import jax.numpy as jnp
import numpy as np
from jax.experimental import pallas as pl


def _ash_block_kernel(x_ref, wt_ref, b_ref, o_ref, *, k: int):
    x = x_ref[...]
    R, D = x.shape

    fill = jnp.sum(x, axis=1, keepdims=True) * (1.0 / k)

    s = jax.lax.bitcast_convert_type(x, jnp.int32)
    key = jnp.where(s < 0, s ^ jnp.int32(0x7FFFFFFF), s)

    hi = (key >> 16).astype(jnp.int16)
    lo = key.astype(jnp.int16) ^ np.int16(-32768)

    def count16(m01):
        a = m01
        while a.shape[1] > 128:
            h = a.shape[1] // 2
            a = a[:, :h] + a[:, h:]
        return jnp.sum(a.astype(jnp.int32), axis=1, keepdims=True)

    def count_ge(v16, t32):
        t16 = jnp.broadcast_to(t32, (R, D)).astype(jnp.int16)
        return count16(jnp.where(v16 >= t16, np.int16(1), np.int16(0)))

    Khi = jnp.full((R, 1), np.int32(-32768))
    for bit in range(15, -1, -1):
        t = Khi + np.int32(1 << bit)
        t = jnp.where(t > 32767, t - 65536, t)
        cnt = count_ge(hi, t)
        Khi = jnp.where(cnt >= k, t, Khi)

    t16 = jnp.broadcast_to(Khi, (R, D)).astype(jnp.int16)
    eq_hi = hi == t16
    cnt_gt_hi = count16(jnp.where(hi > t16, np.int16(1), np.int16(0)))
    k2 = k - cnt_gt_hi
    lo_m = jnp.where(eq_hi, lo, np.int16(-32768))
    Klo = jnp.full((R, 1), np.int32(-32768))
    for bit in range(15, -1, -1):
        t = Klo + np.int32(1 << bit)
        t = jnp.where(t > 32767, t - 65536, t)
        cnt = count_ge(lo_m, t)
        Klo = jnp.where(cnt >= k2, t, Klo)

    Klo16 = jnp.broadcast_to(Klo, (R, D)).astype(jnp.int16)
    gt = (hi > t16) | (eq_hi & (lo_m > Klo16))
    eq = eq_hi & (lo_m == Klo16)
    cnt_gt = count16(jnp.where(gt, np.int16(1), np.int16(0)))
    cnt_eq = count16(jnp.where(eq, np.int16(1), np.int16(0)))
    need = k - cnt_gt

    idx = jax.lax.broadcasted_iota(jnp.int16, (R, D), 1)
    idx_m = jnp.where(eq, idx, np.int16(-1))
    m = cnt_eq - need + 1
    J = jnp.zeros((R, 1), jnp.int32)
    for bit in range(10, -1, -1):
        t = J | np.int32(1 << bit)
        cnt = count_ge(idx_m, t)
        J = jnp.where(cnt >= m, t, J)

    J16 = jnp.broadcast_to(J, (R, D)).astype(jnp.int16)
    mask01 = jnp.where(gt | (eq & (idx <= J16)),
                       np.int16(1), np.int16(0))
    maskf = mask01.astype(jnp.float32)

    colsum = jnp.dot(maskf, wt_ref[...],
                     preferred_element_type=jnp.float32)
    logits = fill * colsum + b_ref[...]
    mx = jnp.max(logits, axis=1, keepdims=True)
    lse = mx + jnp.log(jnp.sum(jnp.exp(logits - mx), axis=1, keepdims=True))
    o_ref[...] = -lse


@jax.jit
def kernel(features, logits, W_fc, b_fc):
    del logits
    B, D = features.shape
    C = W_fc.shape[0]
    k = D - int(round(D * 90 / 100.0))

    R = min(256, B)
    wt = W_fc.T
    b2 = b_fc.reshape(1, C)

    out = pl.pallas_call(
        functools.partial(_ash_block_kernel, k=k),
        grid=(B // R,),
        in_specs=[
            pl.BlockSpec((R, D), lambda i: (i, 0)),
            pl.BlockSpec((D, C), lambda i: (0, 0)),
            pl.BlockSpec((1, C), lambda i: (0, 0)),
        ],
        out_specs=pl.BlockSpec((R, 1), lambda i: (i, 0)),
        out_shape=jax.ShapeDtypeStruct((B, 1), jnp.float32),
    )(features, wt, b2)
    return out


# speedup vs baseline: 52.6901x; 1.4109x over previous
import functools

import jax
import jax.numpy as jnp
import numpy as np
from jax.experimental import pallas as pl


def _ash_block_kernel(x_ref, wt_ref, b_ref, o_ref, *, k: int):
    x = x_ref[...]
    R, D = x.shape
    kf = jnp.float32(k)

    fill = jnp.sum(x, axis=1, keepdims=True) * (1.0 / k)

    s = jax.lax.bitcast_convert_type(x, jnp.int32)
    key = jnp.where(s < 0, s ^ jnp.int32(0x7FFFFFFF), s)

    hi = (key >> 16).astype(jnp.int16)
    lo = key.astype(jnp.int16) ^ np.int16(-32768)

    def count16(m01):
        a = m01
        while a.shape[1] > 128:
            h = a.shape[1] // 2
            a = a[:, :h] + a[:, h:]
        return jnp.sum(a.astype(jnp.float32), axis=1, keepdims=True)

    def count_ge(v16, t32):
        t16 = jnp.broadcast_to(t32.astype(jnp.int16), (R, D))
        return count16(jnp.where(v16 >= t16, np.int16(1), np.int16(0)))

    Khi = jnp.full((R, 1), np.int32(-32768))
    for bit in range(15, -1, -1):
        t = Khi + np.int32(1 << bit)
        cnt = count_ge(hi, t)
        Khi = jnp.where(cnt >= kf, t, Khi)

    t16 = jnp.broadcast_to(Khi.astype(jnp.int16), (R, D))
    eq_hi = hi == t16
    cnt_gt_hi = count16(jnp.where(hi > t16, np.int16(1), np.int16(0)))
    k2 = kf - cnt_gt_hi
    lo_m = jnp.where(eq_hi, lo, np.int16(-32768))
    Klo = jnp.full((R, 1), np.int32(-32768))
    for bit in range(15, -1, -1):
        t = Klo + np.int32(1 << bit)
        cnt = count_ge(lo_m, t)
        Klo = jnp.where(cnt >= k2, t, Klo)

    Klo16 = jnp.broadcast_to(Klo.astype(jnp.int16), (R, D))
    gt = (hi > t16) | (eq_hi & (lo_m > Klo16))
    eq = eq_hi & (lo_m == Klo16)
    cnt_gt = count16(jnp.where(gt, np.int16(1), np.int16(0)))
    cnt_eq = count16(jnp.where(eq, np.int16(1), np.int16(0)))
    need = kf - cnt_gt

    excess_any = jnp.max(cnt_eq - need) > 0.5

    def slow_path(_):
        idx = jax.lax.broadcasted_iota(jnp.int16, (R, D), 1)
        idx_m = jnp.where(eq, idx, np.int16(-1))
        m = cnt_eq - need + 1.0
        J = jnp.zeros((R, 1), jnp.int32)
        for bit in range(10, -1, -1):
            t = J | np.int32(1 << bit)
            cnt = count_ge(idx_m, t)
            J = jnp.where(cnt >= m, t, J)
        J16 = jnp.broadcast_to(J.astype(jnp.int16), (R, D))
        return jnp.where(gt | (eq & (idx <= J16)), np.int16(1), np.int16(0))

    def fast_path(_):
        return jnp.where(gt | eq, np.int16(1), np.int16(0))

    mask01 = jax.lax.cond(excess_any, slow_path, fast_path, operand=None)
    maskf = mask01.astype(jnp.float32)

    colsum = jnp.dot(maskf, wt_ref[...],
                     preferred_element_type=jnp.float32)
    logits = fill * colsum + b_ref[...]
    mx = jnp.max(logits, axis=1, keepdims=True)
    lse = mx + jnp.log(jnp.sum(jnp.exp(logits - mx), axis=1, keepdims=True))
    o_ref[...] = -lse


@jax.jit
def kernel(features, logits, W_fc, b_fc):
    del logits
    B, D = features.shape
    C = W_fc.shape[0]
    k = D - int(round(D * 90 / 100.0))

    R = min(256, B)
    wt = W_fc.T
    b2 = b_fc.reshape(1, C)

    out = pl.pallas_call(
        functools.partial(_ash_block_kernel, k=k),
        grid=(B // R,),
        in_specs=[
            pl.BlockSpec((R, D), lambda i: (i, 0)),
            pl.BlockSpec((D, C), lambda i: (0, 0)),
            pl.BlockSpec((1, C), lambda i: (0, 0)),
        ],
        out_specs=pl.BlockSpec((R, 1), lambda i: (i, 0)),
        out_shape=jax.ShapeDtypeStruct((B, 1), jnp.float32),
    )(features, wt, b2)
    return out
